# baseline (speedup 1.0000x reference)
import jax
import jax.numpy as jnp
from jax import lax
from jax.experimental import pallas as pl
from jax.experimental.pallas import tpu as pltpu
from jax.experimental.pallas import tpu_sc as plsc

N = 10000
E = 320000
D = 128
H = 128
O = 64
G = 64

NC, NS, L = 2, 16, 16
NPAD = 10240
GA = 72
AFLAT = GA * NPAD
ASH = (G + 1) * NPAD
STRIPE = ASH // NS
ZB = STRIPE // 4
ED_T = E // (NC * NS)
DEG_T = E // NS
NODES_T = NPAD // NS
CH = 128
NFULL = ED_T // CH
TAIL = ED_T - NFULL * CH
GRP = 3
DEPTH = 4
NGRP = NFULL // GRP
NBUF = DEPTH * GRP
CNT = 128


def _rsqrt16(d):
    i = plsc.bitcast(d, jnp.int32)
    y = plsc.bitcast(0x5F3759DF - (i >> 1), jnp.float32)
    for _ in range(3):
        y = y * (1.5 - 0.5 * d * y * y)
    return y


def _sc_body(edge, batch,
             a_out, cnt_out, dinv_out,
             a_sh, degstage, dinv_sh, cntstage,
             zbuf, ebuf, fp, bl, acc, tmp2d, cnt80, idx2, val2,
             sem, sem_b, sem_d, sem_s, sem_z):
    c = lax.axis_index("c")
    s = lax.axis_index("s")

    base = s * NODES_T
    ebase = s * DEG_T + c * ED_T
    ld_batch = pltpu.async_copy(batch, bl.at[pl.ds(0, N)], sem_b)
    ld_dst = pltpu.async_copy(edge.at[pl.ds(E + s * DEG_T, DEG_T)],
                              ebuf.at[pl.ds(0, DEG_T)], sem_d)
    ld_src = pltpu.async_copy(edge.at[pl.ds(ebase, ED_T)],
                              ebuf.at[pl.ds(2 * ED_T, ED_T)], sem_s)

    def _z(i, _):
        for u in range(5):
            zbuf[pl.ds((i * 5 + u) * L, L)] = jnp.zeros((L,), jnp.float32)
        return 0
    lax.fori_loop(0, ZB // L // 5, _z, 0)
    for q in range(4):
        pltpu.async_copy(zbuf, a_sh.at[pl.ds(s * STRIPE + q * ZB, ZB)], sem_z)

    def _zf(i, _):
        for u in range(5):
            fp[pl.ds((i * 5 + u) * L, L)] = jnp.zeros((L,), jnp.float32)
        return 0
    lax.fori_loop(0, NPAD // L // 5, _zf, 0)
    for k in range(CNT // L):
        cnt80[pl.ds(k * L, L)] = jnp.zeros((L,), jnp.float32)

    ld_batch.wait()
    for k in range((NPAD - N) // L):
        bl[pl.ds(N + k * L, L)] = jnp.full((L,), G, jnp.int32)

    def _hist(i, _):
        gv = bl[pl.ds(base + i * L, L)]
        plsc.addupdate_scatter(cnt80, [gv], jnp.ones((L,), jnp.float32))
        return 0
    lax.fori_loop(0, NODES_T // L, _hist, 0)
    pltpu.sync_copy(cnt80, cntstage.at[s])

    ld_dst.wait()

    def _deg(i, _):
        for u in range(10):
            dv = ebuf[pl.ds((i * 10 + u) * L, L)]
            plsc.addupdate_scatter(fp, [dv], jnp.ones((L,), jnp.float32))
        return 0
    lax.fori_loop(0, DEG_T // L // 10, _deg, 0)
    pltpu.sync_copy(fp, degstage.at[s])

    ld_src.wait()
    for q in range(4):
        pltpu.make_async_copy(
            zbuf, a_sh.at[pl.ds(s * STRIPE + q * ZB, ZB)], sem_z).wait()
    plsc.subcore_barrier()

    pltpu.sync_copy(degstage.at[:, pl.ds(base, NODES_T)], tmp2d)

    def _red(i, _):
        sl = pl.ds(i * L, L)
        d = tmp2d[0, sl]
        for t in range(1, NS):
            d = d + tmp2d[t, sl]
        acc[sl] = _rsqrt16(d + 1.0)
        return 0
    lax.fori_loop(0, NODES_T // L, _red, 0)
    pltpu.sync_copy(acc, dinv_sh.at[pl.ds(base, NODES_T)])

    @pl.when(c == 0)
    def _():
        pltpu.sync_copy(acc, dinv_out.at[pl.ds(base, NODES_T)])

    @pl.when((c == 0) & (s == 0))
    def _():
        pltpu.sync_copy(cntstage, tmp2d.at[:, pl.ds(0, CNT)])
        for k in range(CNT // L):
            sl = pl.ds(k * L, L)
            v = tmp2d[0, sl]
            for t in range(1, NS):
                v = v + tmp2d[t, sl]
            cnt80[sl] = v
        pltpu.sync_copy(cnt80, cnt_out)
    plsc.subcore_barrier()

    pltpu.sync_copy(dinv_sh, fp)
    dbase = c * ED_T

    def _fill16(off, m, k):
        sv = ebuf[pl.ds(2 * ED_T + off, L)]
        dv = ebuf[pl.ds(dbase + off, L)]
        dd = plsc.load_gather(fp, [dv])
        g = plsc.load_gather(bl, [dv])
        val2[m, pl.ds(k * L, L)] = dd
        idx2[m, pl.ds(k * L, L)] = g * NPAD + sv

    def _drain(n):
        for _ in range(n):
            pltpu.make_async_copy(
                val2.at[0], a_sh.at[pl.ds(0, CH)], sem).wait()

    def _do_group(j, third):
        @pl.when(j >= DEPTH)
        def _():
            _drain(GRP)
        for i in range(GRP):
            for k in range(CH // L):
                _fill16((j * GRP + i) * CH + k * L, third * GRP + i, k)
        for i in range(GRP):
            m = third * GRP + i
            pltpu.async_copy(val2.at[m], a_sh.at[idx2.at[m]], sem, add=True)

    def _edge(j, _):
        for t in range(DEPTH):
            @pl.when(j % DEPTH == t)
            def _(t=t):
                _do_group(j, t)
        return 0
    lax.fori_loop(0, NGRP, _edge, 0)
    _drain(DEPTH * GRP)

    _fill16(NFULL * CH, 0, 0)
    for k in range(TAIL // L, CH // L):
        val2[0, pl.ds(k * L, L)] = jnp.zeros((L,), jnp.float32)
        idx2[0, pl.ds(k * L, L)] = jnp.zeros((L,), jnp.int32)
    pltpu.sync_copy(val2.at[0], a_sh.at[idx2.at[0]], add=True)

    def _self_chunk(j):
        for k in range(CH // L):
            off = base + j * CH + k * L
            iv = jnp.full((L,), off, jnp.int32) + lax.iota(jnp.int32, L)
            y = fp[pl.ds(off, L)]
            g = plsc.load_gather(bl, [iv])
            val2[0, pl.ds(k * L, L)] = y
            idx2[0, pl.ds(k * L, L)] = g * NPAD + iv
        pltpu.sync_copy(val2.at[0], a_sh.at[idx2.at[0]], add=True)

    @pl.when(c == 0)
    def _():
        for j in range(3):
            _self_chunk(j)

    @pl.when(c == 1)
    def _():
        for j in range(3, NODES_T // CH):
            _self_chunk(j)

    plsc.subcore_barrier()

    pltpu.sync_copy(a_sh.at[pl.ds(s * STRIPE, STRIPE)],
                    a_out.at[c, pl.ds(s * STRIPE, STRIPE)])


_sc_call = pl.kernel(
    _sc_body,
    out_type=(jax.ShapeDtypeStruct((NC, AFLAT), jnp.float32),
              jax.ShapeDtypeStruct((CNT,), jnp.float32),
              jax.ShapeDtypeStruct((NPAD,), jnp.float32)),
    mesh=plsc.VectorSubcoreMesh(core_axis_name="c", subcore_axis_name="s",
                                num_cores=NC, num_subcores=NS),
    compiler_params=pltpu.CompilerParams(needs_layout_passes=False),
    scratch_types=[
        pltpu.VMEM_SHARED((ASH,), jnp.float32),
        pltpu.VMEM_SHARED((NS, NPAD), jnp.float32),
        pltpu.VMEM_SHARED((NPAD,), jnp.float32),
        pltpu.VMEM_SHARED((NS, CNT), jnp.float32),
        pltpu.VMEM((ZB,), jnp.float32),
        pltpu.VMEM((3 * ED_T,), jnp.int32),
        pltpu.VMEM((NPAD,), jnp.float32),
        pltpu.VMEM((NPAD,), jnp.int32),
        pltpu.VMEM((NODES_T,), jnp.float32),
        pltpu.VMEM((NS, NODES_T), jnp.float32),
        pltpu.VMEM((CNT,), jnp.float32),
        pltpu.VMEM((NBUF, CH), jnp.int32),
        pltpu.VMEM((NBUF, CH), jnp.float32),
        pltpu.SemaphoreType.DMA,
        pltpu.SemaphoreType.DMA,
        pltpu.SemaphoreType.DMA,
        pltpu.SemaphoreType.DMA,
        pltpu.SemaphoreType.DMA,
    ],
)


def _tc_body(a_ref, x_ref, cnt_ref, dinv_ref, w1_ref, b1_ref, w2_ref,
             b2_ref, o_ref):
    a = (a_ref[:G, :N] + a_ref[GA:GA + G, :N]) * dinv_ref[...]
    p = jnp.dot(a, x_ref[...], preferred_element_type=jnp.float32)
    p = (jnp.dot(p, w1_ref[...], preferred_element_type=jnp.float32)
         + cnt_ref[...][:G].reshape(G, 1) * b1_ref[...].reshape(1, H))
    o_ref[...] = (jnp.dot(p, w2_ref[...], preferred_element_type=jnp.float32)
                  + b2_ref[...].reshape(1, O))


_tc_call = pl.pallas_call(
    _tc_body,
    out_shape=jax.ShapeDtypeStruct((G, O), jnp.float32),
)


@jax.jit
def kernel(x, edge_index, batch, W1, b1, W2, b2):
    a, cnt, dinv = _sc_call(edge_index.reshape(2 * E), batch)
    return _tc_call(a.reshape(NC * GA, NPAD), x, cnt,
                    dinv[:N].reshape(1, N), W1, b1, W2, b2)


# speedup vs baseline: 1.0036x; 1.0036x over previous
import jax
import jax.numpy as jnp
from jax import lax
from jax.experimental import pallas as pl
from jax.experimental.pallas import tpu as pltpu
from jax.experimental.pallas import tpu_sc as plsc

N = 10000
E = 320000
D = 128
H = 128
O = 64
G = 64

NC, NS, L = 2, 16, 16
NPAD = 10240
GA = 72
AFLAT = GA * NPAD
ASH = (G + 1) * NPAD
STRIPE = ASH // NS
ZB = STRIPE // 4
ED_T = E // (NC * NS)
DEG_T = E // NS
NODES_T = NPAD // NS
CH = 128
NFULL = ED_T // CH
TAIL = ED_T - NFULL * CH
GRP = 3
DEPTH = 3
NGRP = NFULL // GRP
NBUF = DEPTH * GRP
CNT = 128


def _rsqrt16(d):
    i = plsc.bitcast(d, jnp.int32)
    y = plsc.bitcast(0x5F3759DF - (i >> 1), jnp.float32)
    for _ in range(3):
        y = y * (1.5 - 0.5 * d * y * y)
    return y


def _sc_body(edge, batch,
             a_out, cnt_out, dinv_out,
             a_sh, degstage, dinv_sh, cntstage,
             zbuf, ebuf, fp, bl, acc, tmp2d, cnt80, idx2, val2,
             sem, sem_b, sem_d, sem_s, sem_z):
    c = lax.axis_index("c")
    s = lax.axis_index("s")

    base = s * NODES_T
    ebase = s * DEG_T + c * ED_T
    ld_batch = pltpu.async_copy(batch, bl.at[pl.ds(0, N)], sem_b)
    ld_dst = pltpu.async_copy(edge.at[pl.ds(E + s * DEG_T, DEG_T)],
                              ebuf.at[pl.ds(0, DEG_T)], sem_d)
    ld_src = pltpu.async_copy(edge.at[pl.ds(ebase, ED_T)],
                              ebuf.at[pl.ds(2 * ED_T, ED_T)], sem_s)

    def _z(i, _):
        for u in range(5):
            zbuf[pl.ds((i * 5 + u) * L, L)] = jnp.zeros((L,), jnp.float32)
        return 0
    lax.fori_loop(0, ZB // L // 5, _z, 0)
    for q in range(4):
        pltpu.async_copy(zbuf, a_sh.at[pl.ds(s * STRIPE + q * ZB, ZB)], sem_z)

    def _zf(i, _):
        for u in range(5):
            fp[pl.ds((i * 5 + u) * L, L)] = jnp.zeros((L,), jnp.float32)
        return 0
    lax.fori_loop(0, NPAD // L // 5, _zf, 0)
    for k in range(CNT // L):
        cnt80[pl.ds(k * L, L)] = jnp.zeros((L,), jnp.float32)

    ld_batch.wait()
    for k in range((NPAD - N) // L):
        bl[pl.ds(N + k * L, L)] = jnp.full((L,), G, jnp.int32)

    def _hist(i, _):
        gv = bl[pl.ds(base + i * L, L)]
        plsc.addupdate_scatter(cnt80, [gv], jnp.ones((L,), jnp.float32))
        return 0
    lax.fori_loop(0, NODES_T // L, _hist, 0)
    pltpu.sync_copy(cnt80, cntstage.at[s])

    ld_dst.wait()

    def _deg(i, _):
        for u in range(10):
            dv = ebuf[pl.ds((i * 10 + u) * L, L)]
            plsc.addupdate_scatter(fp, [dv], jnp.ones((L,), jnp.float32))
        return 0
    lax.fori_loop(0, DEG_T // L // 10, _deg, 0)
    pltpu.sync_copy(fp, degstage.at[s])

    ld_src.wait()
    for q in range(4):
        pltpu.make_async_copy(
            zbuf, a_sh.at[pl.ds(s * STRIPE + q * ZB, ZB)], sem_z).wait()
    plsc.subcore_barrier()

    pltpu.sync_copy(degstage.at[:, pl.ds(base, NODES_T)], tmp2d)

    def _red(i, _):
        sl = pl.ds(i * L, L)
        d = tmp2d[0, sl]
        for t in range(1, NS):
            d = d + tmp2d[t, sl]
        acc[sl] = _rsqrt16(d + 1.0)
        return 0
    lax.fori_loop(0, NODES_T // L, _red, 0)
    pltpu.sync_copy(acc, dinv_sh.at[pl.ds(base, NODES_T)])

    @pl.when(c == 0)
    def _():
        pltpu.sync_copy(acc, dinv_out.at[pl.ds(base, NODES_T)])

    @pl.when((c == 0) & (s == 0))
    def _():
        pltpu.sync_copy(cntstage, cnt_out)
    plsc.subcore_barrier()

    pltpu.sync_copy(dinv_sh, fp)
    dbase = c * ED_T

    def _fill16(off, m, k):
        sv = ebuf[pl.ds(2 * ED_T + off, L)]
        dv = ebuf[pl.ds(dbase + off, L)]
        dd = plsc.load_gather(fp, [dv])
        g = plsc.load_gather(bl, [dv])
        val2[m, pl.ds(k * L, L)] = dd
        idx2[m, pl.ds(k * L, L)] = g * NPAD + sv

    def _drain(n):
        for _ in range(n):
            pltpu.make_async_copy(
                val2.at[0], a_sh.at[pl.ds(0, CH)], sem).wait()

    def _do_group(j, third):
        @pl.when(j >= DEPTH)
        def _():
            _drain(GRP)
        for i in range(GRP):
            for k in range(CH // L):
                _fill16((j * GRP + i) * CH + k * L, third * GRP + i, k)
        for i in range(GRP):
            m = third * GRP + i
            pltpu.async_copy(val2.at[m], a_sh.at[idx2.at[m]], sem, add=True)

    def _edge(j, _):
        for t in range(DEPTH):
            @pl.when(j % DEPTH == t)
            def _(t=t):
                _do_group(j, t)
        return 0
    lax.fori_loop(0, NGRP, _edge, 0)
    _drain(DEPTH * GRP)

    _fill16(NFULL * CH, 0, 0)
    for k in range(TAIL // L, CH // L):
        val2[0, pl.ds(k * L, L)] = jnp.zeros((L,), jnp.float32)
        idx2[0, pl.ds(k * L, L)] = jnp.zeros((L,), jnp.int32)
    pltpu.sync_copy(val2.at[0], a_sh.at[idx2.at[0]], add=True)

    def _self_chunk(j):
        for k in range(CH // L):
            off = base + j * CH + k * L
            iv = jnp.full((L,), off, jnp.int32) + lax.iota(jnp.int32, L)
            y = fp[pl.ds(off, L)]
            g = plsc.load_gather(bl, [iv])
            val2[0, pl.ds(k * L, L)] = y
            idx2[0, pl.ds(k * L, L)] = g * NPAD + iv
        pltpu.sync_copy(val2.at[0], a_sh.at[idx2.at[0]], add=True)

    @pl.when(c == 0)
    def _():
        for j in range(3):
            _self_chunk(j)

    @pl.when(c == 1)
    def _():
        for j in range(3, NODES_T // CH):
            _self_chunk(j)

    plsc.subcore_barrier()

    pltpu.sync_copy(a_sh.at[pl.ds(s * STRIPE, STRIPE)],
                    a_out.at[c, pl.ds(s * STRIPE, STRIPE)])


_sc_call = pl.kernel(
    _sc_body,
    out_type=(jax.ShapeDtypeStruct((NC, AFLAT), jnp.float32),
              jax.ShapeDtypeStruct((NS, CNT), jnp.float32),
              jax.ShapeDtypeStruct((NPAD,), jnp.float32)),
    mesh=plsc.VectorSubcoreMesh(core_axis_name="c", subcore_axis_name="s",
                                num_cores=NC, num_subcores=NS),
    compiler_params=pltpu.CompilerParams(needs_layout_passes=False),
    scratch_types=[
        pltpu.VMEM_SHARED((ASH,), jnp.float32),
        pltpu.VMEM_SHARED((NS, NPAD), jnp.float32),
        pltpu.VMEM_SHARED((NPAD,), jnp.float32),
        pltpu.VMEM_SHARED((NS, CNT), jnp.float32),
        pltpu.VMEM((ZB,), jnp.float32),
        pltpu.VMEM((3 * ED_T,), jnp.int32),
        pltpu.VMEM((NPAD,), jnp.float32),
        pltpu.VMEM((NPAD,), jnp.int32),
        pltpu.VMEM((NODES_T,), jnp.float32),
        pltpu.VMEM((NS, NODES_T), jnp.float32),
        pltpu.VMEM((CNT,), jnp.float32),
        pltpu.VMEM((NBUF, CH), jnp.int32),
        pltpu.VMEM((NBUF, CH), jnp.float32),
        pltpu.SemaphoreType.DMA,
        pltpu.SemaphoreType.DMA,
        pltpu.SemaphoreType.DMA,
        pltpu.SemaphoreType.DMA,
        pltpu.SemaphoreType.DMA,
    ],
)


def _tc_body(a_ref, x_ref, cnt_ref, dinv_ref, w1_ref, b1_ref, w2_ref,
             b2_ref, o_ref):
    a = (a_ref[:G, :N] + a_ref[GA:GA + G, :N]) * dinv_ref[...]
    p = jnp.dot(a, x_ref[...], preferred_element_type=jnp.float32)
    p = (jnp.dot(p, w1_ref[...], preferred_element_type=jnp.float32)
         + jnp.sum(cnt_ref[...], axis=0)[:G].reshape(G, 1)
         * b1_ref[...].reshape(1, H))
    o_ref[...] = (jnp.dot(p, w2_ref[...], preferred_element_type=jnp.float32)
                  + b2_ref[...].reshape(1, O))


_tc_call = pl.pallas_call(
    _tc_body,
    out_shape=jax.ShapeDtypeStruct((G, O), jnp.float32),
)


@jax.jit
def kernel(x, edge_index, batch, W1, b1, W2, b2):
    a, cnt, dinv = _sc_call(edge_index.reshape(2 * E), batch)
    return _tc_call(a.reshape(NC * GA, NPAD), x, cnt,
                    dinv[:N].reshape(1, N), W1, b1, W2, b2)


# speedup vs baseline: 1.0131x; 1.0094x over previous
import jax
import jax.numpy as jnp
from jax import lax
from jax.experimental import pallas as pl
from jax.experimental.pallas import tpu as pltpu
from jax.experimental.pallas import tpu_sc as plsc

N = 10000
E = 320000
D = 128
H = 128
O = 64
G = 64

NC, NS, L = 2, 16, 16
NPAD = 10240
GA = 72
AFLAT = GA * NPAD
ASH = (G + 1) * NPAD
STRIPE = ASH // NS
ZB = STRIPE // 4
ED_T = E // (NC * NS)
DEG_T = E // NS
NODES_T = NPAD // NS
CH = 128
NFULL = ED_T // CH
TAIL = ED_T - NFULL * CH
GRP = 3
DEPTH = 3
NGRP = NFULL // GRP
NBUF = DEPTH * GRP
CNT = 128


def _rsqrt16(d):
    i = plsc.bitcast(d, jnp.int32)
    y = plsc.bitcast(0x5F3759DF - (i >> 1), jnp.float32)
    for _ in range(3):
        y = y * (1.5 - 0.5 * d * y * y)
    return y


def _sc_body(edge, batch,
             a_out, cnt_out, dinv_out,
             a_sh, degstage, dinv_sh, cntstage,
             zbuf, ebuf, fp, bl, acc, tmp2d, cnt80, idx2, val2,
             sem, sem_b, sem_d, sem_s, sem_z):
    c = lax.axis_index("c")
    s = lax.axis_index("s")

    base = s * NODES_T
    ebase = s * DEG_T + c * ED_T
    ld_batch = pltpu.async_copy(batch, bl.at[pl.ds(0, N)], sem_b)
    ld_dst = pltpu.async_copy(edge.at[pl.ds(E + s * DEG_T, DEG_T)],
                              ebuf.at[pl.ds(0, DEG_T)], sem_d)
    ld_src = pltpu.async_copy(edge.at[pl.ds(ebase, ED_T)],
                              ebuf.at[pl.ds(2 * ED_T, ED_T)], sem_s)

    def _z(i, _):
        for u in range(5):
            zbuf[pl.ds((i * 5 + u) * L, L)] = jnp.zeros((L,), jnp.float32)
        return 0
    lax.fori_loop(0, ZB // L // 5, _z, 0)
    for q in range(4):
        pltpu.async_copy(zbuf, a_sh.at[pl.ds(s * STRIPE + q * ZB, ZB)], sem_z)

    def _zf(i, _):
        for u in range(5):
            fp[pl.ds((i * 5 + u) * L, L)] = jnp.zeros((L,), jnp.float32)
        return 0
    lax.fori_loop(0, NPAD // L // 5, _zf, 0)
    for k in range(CNT // L):
        cnt80[pl.ds(k * L, L)] = jnp.zeros((L,), jnp.float32)

    ld_batch.wait()
    for k in range((NPAD - N) // L):
        bl[pl.ds(N + k * L, L)] = jnp.full((L,), G, jnp.int32)

    def _hist(i, _):
        gv = bl[pl.ds(base + i * L, L)]
        plsc.addupdate_scatter(cnt80, [gv], jnp.ones((L,), jnp.float32))
        return 0
    lax.fori_loop(0, NODES_T // L, _hist, 0)
    pltpu.sync_copy(cnt80, cntstage.at[s])

    ld_dst.wait()

    def _deg(i, _):
        for u in range(10):
            dv = ebuf[pl.ds((i * 10 + u) * L, L)]
            plsc.addupdate_scatter(fp, [dv], jnp.ones((L,), jnp.float32))
        return 0
    lax.fori_loop(0, DEG_T // L // 10, _deg, 0)
    pltpu.sync_copy(fp, degstage.at[s])

    ld_src.wait()
    for q in range(4):
        pltpu.make_async_copy(
            zbuf, a_sh.at[pl.ds(s * STRIPE + q * ZB, ZB)], sem_z).wait()
    plsc.subcore_barrier()

    pltpu.sync_copy(degstage.at[:, pl.ds(base, NODES_T)], tmp2d)

    def _red(i, _):
        sl = pl.ds(i * L, L)
        d = tmp2d[0, sl]
        for t in range(1, NS):
            d = d + tmp2d[t, sl]
        acc[sl] = _rsqrt16(d + 1.0)
        return 0
    lax.fori_loop(0, NODES_T // L, _red, 0)
    pltpu.sync_copy(acc, dinv_sh.at[pl.ds(base, NODES_T)])

    @pl.when(c == 0)
    def _():
        pltpu.sync_copy(acc, dinv_out.at[0, pl.ds(base, NODES_T)])

    @pl.when((c == 0) & (s == 0))
    def _():
        pltpu.sync_copy(cntstage, cnt_out)
    plsc.subcore_barrier()

    pltpu.sync_copy(dinv_sh, fp)
    dbase = c * ED_T

    def _fill16(off, m, k):
        sv = ebuf[pl.ds(2 * ED_T + off, L)]
        dv = ebuf[pl.ds(dbase + off, L)]
        dd = plsc.load_gather(fp, [dv])
        g = plsc.load_gather(bl, [dv])
        val2[m, pl.ds(k * L, L)] = dd
        idx2[m, pl.ds(k * L, L)] = g * NPAD + sv

    def _drain(n):
        for _ in range(n):
            pltpu.make_async_copy(
                val2.at[0], a_sh.at[pl.ds(0, CH)], sem).wait()

    def _do_group(j, third):
        @pl.when(j >= DEPTH)
        def _():
            _drain(GRP)
        for i in range(GRP):
            for k in range(CH // L):
                _fill16((j * GRP + i) * CH + k * L, third * GRP + i, k)
        for i in range(GRP):
            m = third * GRP + i
            pltpu.async_copy(val2.at[m], a_sh.at[idx2.at[m]], sem, add=True)

    def _edge(j, _):
        for t in range(DEPTH):
            @pl.when(j % DEPTH == t)
            def _(t=t):
                _do_group(j, t)
        return 0
    lax.fori_loop(0, NGRP, _edge, 0)
    _drain(DEPTH * GRP)

    _fill16(NFULL * CH, 0, 0)
    for k in range(TAIL // L, CH // L):
        val2[0, pl.ds(k * L, L)] = jnp.zeros((L,), jnp.float32)
        idx2[0, pl.ds(k * L, L)] = jnp.zeros((L,), jnp.int32)
    pltpu.sync_copy(val2.at[0], a_sh.at[idx2.at[0]], add=True)

    def _self_chunk(j):
        for k in range(CH // L):
            off = base + j * CH + k * L
            iv = jnp.full((L,), off, jnp.int32) + lax.iota(jnp.int32, L)
            y = fp[pl.ds(off, L)]
            g = plsc.load_gather(bl, [iv])
            val2[0, pl.ds(k * L, L)] = y
            idx2[0, pl.ds(k * L, L)] = g * NPAD + iv
        pltpu.sync_copy(val2.at[0], a_sh.at[idx2.at[0]], add=True)

    @pl.when(c == 0)
    def _():
        for j in range(3):
            _self_chunk(j)

    @pl.when(c == 1)
    def _():
        for j in range(3, NODES_T // CH):
            _self_chunk(j)

    plsc.subcore_barrier()

    pltpu.sync_copy(a_sh.at[pl.ds(s * STRIPE, STRIPE)],
                    a_out.at[c, pl.ds(s * STRIPE, STRIPE)])


_sc_call = pl.kernel(
    _sc_body,
    out_type=(jax.ShapeDtypeStruct((NC, AFLAT), jnp.float32),
              jax.ShapeDtypeStruct((NS, CNT), jnp.float32),
              jax.ShapeDtypeStruct((1, NPAD), jnp.float32)),
    mesh=plsc.VectorSubcoreMesh(core_axis_name="c", subcore_axis_name="s",
                                num_cores=NC, num_subcores=NS),
    compiler_params=pltpu.CompilerParams(needs_layout_passes=False),
    scratch_types=[
        pltpu.VMEM_SHARED((ASH,), jnp.float32),
        pltpu.VMEM_SHARED((NS, NPAD), jnp.float32),
        pltpu.VMEM_SHARED((NPAD,), jnp.float32),
        pltpu.VMEM_SHARED((NS, CNT), jnp.float32),
        pltpu.VMEM((ZB,), jnp.float32),
        pltpu.VMEM((3 * ED_T,), jnp.int32),
        pltpu.VMEM((NPAD,), jnp.float32),
        pltpu.VMEM((NPAD,), jnp.int32),
        pltpu.VMEM((NODES_T,), jnp.float32),
        pltpu.VMEM((NS, NODES_T), jnp.float32),
        pltpu.VMEM((CNT,), jnp.float32),
        pltpu.VMEM((NBUF, CH), jnp.int32),
        pltpu.VMEM((NBUF, CH), jnp.float32),
        pltpu.SemaphoreType.DMA,
        pltpu.SemaphoreType.DMA,
        pltpu.SemaphoreType.DMA,
        pltpu.SemaphoreType.DMA,
        pltpu.SemaphoreType.DMA,
    ],
)


def _tc_body(a_ref, x_ref, cnt_ref, dinv_ref, w1_ref, b1_ref, w2_ref,
             b2_ref, o_ref):
    a = (a_ref[:G, :N] + a_ref[GA:GA + G, :N]) * dinv_ref[:, :N]
    p = jnp.dot(a, x_ref[...], preferred_element_type=jnp.float32)
    p = (jnp.dot(p, w1_ref[...], preferred_element_type=jnp.float32)
         + jnp.sum(cnt_ref[...], axis=0)[:G].reshape(G, 1)
         * b1_ref[...].reshape(1, H))
    o_ref[...] = (jnp.dot(p, w2_ref[...], preferred_element_type=jnp.float32)
                  + b2_ref[...].reshape(1, O))


_tc_call = pl.pallas_call(
    _tc_body,
    out_shape=jax.ShapeDtypeStruct((G, O), jnp.float32),
)


@jax.jit
def kernel(x, edge_index, batch, W1, b1, W2, b2):
    a, cnt, dinv = _sc_call(edge_index.reshape(2 * E), batch)
    return _tc_call(a.reshape(NC * GA, NPAD), x, cnt, dinv, W1, b1, W2, b2)
